# baseline (speedup 1.0000x reference)
import functools

import jax
import jax.numpy as jnp
from jax import lax
from jax.experimental import pallas as pl
from jax.experimental.pallas import tpu as pltpu
from jax.experimental.pallas import tpu_sc as plsc

_INFO = plsc.get_sparse_core_info()
_NC, _NS = _INFO.num_cores, _INFO.num_subcores
_NW = _NC * _NS
_L = 16

_CHUNK = 128


def _gather_kernel(b_per_w, T, D, b_dim, j_hbm, h_hbm, tab_hbm, out_hbm,
                   jbuf, hbuf, g0, g1, t0b, t1b, gs0, gs1, ws0, ws1):
  gbufs, tbufs = (g0, g1), (t0b, t1b)
  gsems, wsems = (gs0, gs1), (ws0, ws1)
  wid = lax.axis_index("s") * _NC + lax.axis_index("c")
  base = wid * b_per_w
  pltpu.sync_copy(j_hbm.at[pl.ds(base, b_per_w)], jbuf)
  pltpu.sync_copy(h_hbm.at[pl.ds(base, b_per_w)], hbuf)

  iota = lax.iota(jnp.int32, _L)

  def fire(t, p):
    off = pl.multiple_of(t * _CHUNK, _CHUNK)
    pltpu.async_copy(tab_hbm.at[jbuf.at[pl.ds(off, _CHUNK)]],
                     gbufs[p], gsems[p])

  def wait_g(p):
    pltpu.make_async_copy(tab_hbm.at[pl.ds(0, _CHUNK)], gbufs[p],
                          gsems[p]).wait()

  def compute(t, p):
    hoff = pl.multiple_of(t * _CHUNK, _CHUNK)

    @plsc.parallel_loop(0, _CHUNK, step=1, unroll=8)
    def _tok(tk):
      tvec = iota * 0 + tk
      hvec = plsc.load_gather(hbuf, [tvec + hoff])
      for d0 in range(0, D, _L):
        vals = plsc.load_gather(gbufs[p], [tvec, hvec + (iota + d0)])
        plsc.store_scatter(tbufs[p], [iota + d0, tvec], vals)

  def awrite(t, p):
    off = base + t * _CHUNK
    s = lax.shift_right_logical(off, 12)
    b0 = pl.multiple_of(lax.bitwise_and(off, b_dim - 1), _CHUNK)
    pltpu.async_copy(tbufs[p].at[:, pl.ds(0, _CHUNK)],
                     out_hbm.at[s, :, pl.ds(b0, _CHUNK)], wsems[p])

  def wait_w(p):
    pltpu.make_async_copy(tbufs[p].at[:, pl.ds(0, _CHUNK)],
                          out_hbm.at[0, :, pl.ds(0, _CHUNK)],
                          wsems[p]).wait()

  fire(0, 0)
  fire(1, 1)
  wait_g(0); compute(0, 0); awrite(0, 0); fire(2, 0)
  wait_g(1); compute(1, 1); awrite(1, 1); fire(3, 1)

  @pl.loop(2, T - 2, step=2)
  def _main(t):
    for p in range(2):
      wait_g(p)
      wait_w(p)
      compute(t + p, p)
      awrite(t + p, p)
      fire(t + p + 2, p)

  wait_g(0); wait_w(0); compute(T - 2, 0); awrite(T - 2, 0)
  wait_g(1); wait_w(1); compute(T - 1, 1); awrite(T - 1, 1)
  wait_w(0); wait_w(1)


@jax.jit
def kernel(input, table):
  b_dim, s_dim = input.shape
  v_dim, d_dim = table.shape
  n = b_dim * s_dim

  idx_t = input.T.reshape(-1).astype(jnp.int32)
  j_t = idx_t >> 1
  h_t = (idx_t & 1) << 6

  tab2 = table.reshape(v_dim // 2, 2 * d_dim)

  assert n % (_NW * _CHUNK) == 0 and b_dim % _CHUNK == 0
  b_per_w = n // _NW
  T = b_per_w // _CHUNK
  assert T % 2 == 0 and (T - 4) % 2 == 0

  mesh = plsc.VectorSubcoreMesh(core_axis_name="c", subcore_axis_name="s")
  out = pl.kernel(
      functools.partial(_gather_kernel, b_per_w, T, d_dim, b_dim),
      out_type=jax.ShapeDtypeStruct((s_dim, d_dim, b_dim), jnp.float32),
      mesh=mesh,
      scratch_types=[
          pltpu.VMEM((b_per_w,), jnp.int32),
          pltpu.VMEM((b_per_w,), jnp.int32),
          pltpu.VMEM((_CHUNK, 2 * d_dim), jnp.float32),
          pltpu.VMEM((_CHUNK, 2 * d_dim), jnp.float32),
          pltpu.VMEM((d_dim, _CHUNK + 1), jnp.float32),
          pltpu.VMEM((d_dim, _CHUNK + 1), jnp.float32),
          pltpu.SemaphoreType.DMA,
          pltpu.SemaphoreType.DMA,
          pltpu.SemaphoreType.DMA,
          pltpu.SemaphoreType.DMA,
      ],
      compiler_params=pltpu.CompilerParams(use_tc_tiling_on_sc=True,
                                           needs_layout_passes=False),
  )(j_t, h_t, tab2)
  return out.transpose(2, 0, 1)


# speedup vs baseline: 1.1454x; 1.1454x over previous
import functools

import jax
import jax.numpy as jnp
from jax import lax
from jax.experimental import pallas as pl
from jax.experimental.pallas import tpu as pltpu
from jax.experimental.pallas import tpu_sc as plsc

EMBED_DIM = 64

_INFO = plsc.get_sparse_core_info()
_NC, _NS = _INFO.num_cores, _INFO.num_subcores
_NW = _NC * _NS

_NT = 128
_CH = 256
_NBUF = 4


def _gather_kernel(b_per_w, T, idx_hbm, table_hbm, out_hbm, idx_v,
                   b0, b1, b2, b3, g0, g1, g2, g3, w0, w1, w2, w3):
  bufs = (b0, b1, b2, b3)
  gsems = (g0, g1, g2, g3)
  wsems = (w0, w1, w2, w3)
  wid = lax.axis_index("s") * _NC + lax.axis_index("c")
  base = wid * b_per_w
  pltpu.sync_copy(idx_hbm.at[pl.ds(base, b_per_w)], idx_v)

  def fire(t, b):
    for j in range(_CH // _NT):
      off = pl.multiple_of(t * _CH + j * _NT, _NT)
      pltpu.async_copy(table_hbm.at[idx_v.at[pl.ds(off, _NT)]],
                       bufs[b].at[pl.ds(j * _NT, _NT)], gsems[b])

  def wait_g(b):
    pltpu.make_async_copy(table_hbm.at[pl.ds(0, _CH)], bufs[b],
                          gsems[b]).wait()

  def awrite(t, b):
    off = pl.multiple_of(base + t * _CH, _NT)
    pltpu.async_copy(bufs[b], out_hbm.at[pl.ds(off, _CH)], wsems[b])

  def wait_w(b):
    pltpu.make_async_copy(bufs[b], out_hbm.at[pl.ds(0, _CH)],
                          wsems[b]).wait()

  fire(0, 0)
  fire(1, 1)
  wait_g(0); awrite(0, 0); fire(2, 2)
  wait_g(1); awrite(1, 1); fire(3, 3)

  @pl.loop(2, T - 2, step=_NBUF)
  def _main(t0):
    for i in range(_NBUF):
      t = t0 + i
      b = (2 + i) % _NBUF
      wait_g(b)
      awrite(t, b)
      wait_w((b + 2) % _NBUF)
      fire(t + 2, (b + 2) % _NBUF)

  wait_g(2); awrite(T - 2, 2); wait_w(0)
  wait_g(3); awrite(T - 1, 3); wait_w(1)
  wait_w(2); wait_w(3)


@jax.jit
def kernel(input, table):
  orig_shape = input.shape
  flat_idx = input.reshape(-1).astype(jnp.int32)
  n = flat_idx.shape[0]
  assert n % (_NW * _CH) == 0
  b_per_w = n // _NW
  T = b_per_w // _CH
  assert (T - 4) % _NBUF == 0

  mesh = plsc.VectorSubcoreMesh(core_axis_name="c", subcore_axis_name="s")
  out = pl.kernel(
      functools.partial(_gather_kernel, b_per_w, T),
      out_type=jax.ShapeDtypeStruct((n, EMBED_DIM), jnp.float32),
      mesh=mesh,
      scratch_types=[pltpu.VMEM((b_per_w,), jnp.int32)]
      + [pltpu.VMEM((_CH, EMBED_DIM), jnp.float32) for _ in range(_NBUF)]
      + [pltpu.SemaphoreType.DMA for _ in range(2 * _NBUF)],
      compiler_params=pltpu.CompilerParams(use_tc_tiling_on_sc=False),
  )(flat_idx, table)
  return out.reshape(*orig_shape, EMBED_DIM)
